# Initial kernel scaffold; baseline (speedup 1.0000x reference)
import jax
import jax.numpy as jnp
from jax.experimental import pallas as pl


def kernel(x, Slist, down0_mixer, down0_weight, down0_bias, down1_mixer, down1_weight, down1_bias, up0_mixer, up0_weight, up0_bias, up1_mixer, up1_weight, up1_bias, sc0_mixer, sc0_weight, sc0_bias, sc1_mixer, sc1_weight, sc1_bias):
    raise NotImplementedError("write your pallas kernel here")


# speedup vs baseline: 1.6152x; 1.6152x over previous
import functools

import jax
import jax.numpy as jnp
from jax.experimental import pallas as pl
from jax.experimental.pallas import tpu as pltpu

_NEG = -1e9


def _layer(xt, mask, mc, w0, w1, brow):
    y = jnp.dot(xt, mc, preferred_element_type=jnp.float32)
    outs = []
    for p in range(2):
        y1 = y[:, p:p + 1]
        y2 = jnp.transpose(y[:, 2 + p:3 + p])
        e = y1 + y2
        e = jnp.where(e >= 0, e, 0.2 * e)
        s = jnp.where(mask, e, _NEG)
        m = jnp.max(s, axis=1, keepdims=True)
        ex = jnp.exp(s - m)
        a = ex * (1.0 / jnp.sum(ex, axis=1, keepdims=True))
        z = jnp.dot(a, xt, preferred_element_type=jnp.float32)
        o = (jnp.dot(xt, w0[p], preferred_element_type=jnp.float32)
             + jnp.dot(z, w1[p], preferred_element_type=jnp.float32)
             + brow)
        outs.append(jnp.maximum(o, 0.0))
    return jnp.concatenate(outs, axis=1)


def _body(xt_ref, s_ref,
          mc0, w00, w10, b0,
          mc1, w01, w11, b1,
          mc2, w02, w12, b2,
          mc3, w03, w13, b3,
          mc4, w04, w14, b4,
          mc5, w05, w15, b5,
          out_ref):
    xt = xt_ref[0]
    mask0 = jnp.abs(s_ref[0, 0]) > 1e-9
    mask1 = jnp.abs(s_ref[0, 1]) > 1e-9
    p1 = _layer(xt, mask0, mc0[...], w00[...], w10[...], b0[...])
    p2 = _layer(p1, mask1, mc1[...], w01[...], w11[...], b1[...])
    p3 = (_layer(p2, mask1, mc2[...], w02[...], w12[...], b2[...])
          + _layer(p1, mask1, mc5[...], w05[...], w15[...], b5[...]))
    p4 = (_layer(p3, mask0, mc3[...], w03[...], w13[...], b3[...])
          + _layer(xt, mask0, mc4[...], w04[...], w14[...], b4[...]))
    out_ref[0] = p4


def _prep(mixer, weight, bias):
    g = mixer.shape[2] // 2
    a1 = mixer[:, 0, :g]
    a2 = mixer[:, 0, g:]
    mc = jnp.concatenate([a1, a2], axis=0).T
    w0 = weight[:, 0, 0]
    w1 = weight[:, 0, 1]
    brow = bias.T
    return mc, w0, w1, brow


@jax.jit
def kernel(x, Slist,
           down0_mixer, down0_weight, down0_bias,
           down1_mixer, down1_weight, down1_bias,
           up0_mixer, up0_weight, up0_bias,
           up1_mixer, up1_weight, up1_bias,
           sc0_mixer, sc0_weight, sc0_bias,
           sc1_mixer, sc1_weight, sc1_bias):
    B, Fin, N = x.shape
    xt = jnp.transpose(x, (0, 2, 1))

    params = []
    for m, w, b in ((down0_mixer, down0_weight, down0_bias),
                    (down1_mixer, down1_weight, down1_bias),
                    (up0_mixer, up0_weight, up0_bias),
                    (up1_mixer, up1_weight, up1_bias),
                    (sc0_mixer, sc0_weight, sc0_bias),
                    (sc1_mixer, sc1_weight, sc1_bias)):
        params.extend(_prep(m, w, b))

    full = lambda a: pl.BlockSpec(a.shape, lambda b: (0,) * a.ndim)
    in_specs = [
        pl.BlockSpec((1, N, Fin), lambda b: (b, 0, 0)),
        pl.BlockSpec((1, 2, N, N), lambda b: (b, 0, 0, 0)),
    ] + [full(p) for p in params]

    out = pl.pallas_call(
        _body,
        grid=(B,),
        in_specs=in_specs,
        out_specs=pl.BlockSpec((1, N, 2 * Fin), lambda b: (b, 0, 0)),
        out_shape=jax.ShapeDtypeStruct((B, N, 2 * Fin), jnp.float32),
        compiler_params=pltpu.CompilerParams(
            dimension_semantics=("arbitrary",),
        ),
    )(xt, Slist, *params)

    return jnp.transpose(out, (0, 2, 1))


# speedup vs baseline: 1.7431x; 1.0792x over previous
import functools

import jax
import jax.numpy as jnp
from jax.experimental import pallas as pl
from jax.experimental.pallas import tpu as pltpu

_NEG = -1e9


def _layer(xt, maskf, mc, w0, w1, brow):
    y = jnp.dot(xt, mc, preferred_element_type=jnp.float32)
    xtb = xt.astype(jnp.bfloat16)
    outs = []
    for p in range(2):
        y1 = y[:, p:p + 1]
        y2c = y[:, 2 + p:3 + p]
        y2 = jnp.transpose(y2c)
        y2max = jnp.max(y2c)
        vb = y1 + y2max
        mrow = jnp.maximum(vb, 0.2 * vb)
        v = y1 + y2
        e = jnp.maximum(v, 0.2 * v)
        ex = jnp.exp(e - mrow) * maskf
        a = ex * (1.0 / jnp.sum(ex, axis=1, keepdims=True))
        z = jnp.dot(a.astype(jnp.bfloat16), xtb,
                    preferred_element_type=jnp.float32)
        o = (jnp.dot(xtb, w0[p], preferred_element_type=jnp.float32)
             + jnp.dot(z.astype(jnp.bfloat16), w1[p],
                       preferred_element_type=jnp.float32)
             + brow)
        outs.append(jnp.maximum(o, 0.0))
    return jnp.concatenate(outs, axis=1)


def _body(xt_ref, s_ref,
          mc0, w00, w10, b0,
          mc1, w01, w11, b1,
          mc2, w02, w12, b2,
          mc3, w03, w13, b3,
          mc4, w04, w14, b4,
          mc5, w05, w15, b5,
          out_ref):
    xt = xt_ref[0]
    mask0 = (jnp.abs(s_ref[0, 0]) > 1e-9).astype(jnp.float32)
    mask1 = (jnp.abs(s_ref[0, 1]) > 1e-9).astype(jnp.float32)
    p1 = _layer(xt, mask0, mc0[...], w00[...], w10[...], b0[...])
    p2 = _layer(p1, mask1, mc1[...], w01[...], w11[...], b1[...])
    p3 = (_layer(p2, mask1, mc2[...], w02[...], w12[...], b2[...])
          + _layer(p1, mask1, mc5[...], w05[...], w15[...], b5[...]))
    p4 = (_layer(p3, mask0, mc3[...], w03[...], w13[...], b3[...])
          + _layer(xt, mask0, mc4[...], w04[...], w14[...], b4[...]))
    out_ref[0] = p4


def _prep(mixer, weight, bias):
    g = mixer.shape[2] // 2
    a1 = mixer[:, 0, :g]
    a2 = mixer[:, 0, g:]
    mc = jnp.concatenate([a1, a2], axis=0).T
    w0 = weight[:, 0, 0].astype(jnp.bfloat16)
    w1 = weight[:, 0, 1].astype(jnp.bfloat16)
    brow = bias.T
    return mc, w0, w1, brow


@jax.jit
def kernel(x, Slist,
           down0_mixer, down0_weight, down0_bias,
           down1_mixer, down1_weight, down1_bias,
           up0_mixer, up0_weight, up0_bias,
           up1_mixer, up1_weight, up1_bias,
           sc0_mixer, sc0_weight, sc0_bias,
           sc1_mixer, sc1_weight, sc1_bias):
    B, Fin, N = x.shape
    xt = jnp.transpose(x, (0, 2, 1))

    params = []
    for m, w, b in ((down0_mixer, down0_weight, down0_bias),
                    (down1_mixer, down1_weight, down1_bias),
                    (up0_mixer, up0_weight, up0_bias),
                    (up1_mixer, up1_weight, up1_bias),
                    (sc0_mixer, sc0_weight, sc0_bias),
                    (sc1_mixer, sc1_weight, sc1_bias)):
        params.extend(_prep(m, w, b))

    full = lambda a: pl.BlockSpec(a.shape, lambda b: (0,) * a.ndim)
    in_specs = [
        pl.BlockSpec((1, N, Fin), lambda b: (b, 0, 0)),
        pl.BlockSpec((1, 2, N, N), lambda b: (b, 0, 0, 0)),
    ] + [full(p) for p in params]

    out = pl.pallas_call(
        _body,
        grid=(B,),
        in_specs=in_specs,
        out_specs=pl.BlockSpec((1, N, 2 * Fin), lambda b: (b, 0, 0)),
        out_shape=jax.ShapeDtypeStruct((B, N, 2 * Fin), jnp.float32),
        compiler_params=pltpu.CompilerParams(
            dimension_semantics=("parallel",),
        ),
    )(xt, Slist, *params)

    return jnp.transpose(out, (0, 2, 1))


# speedup vs baseline: 2.4358x; 1.3974x over previous
import functools

import jax
import jax.numpy as jnp
from jax.experimental import pallas as pl
from jax.experimental.pallas import tpu as pltpu

_LOG2E = 1.4426950408889634


def _layer(xt, maskf, mc, w0, w1, brow):
    n, g = xt.shape
    y = jnp.dot(xt, mc, preferred_element_type=jnp.float32)
    xtb = xt.astype(jnp.bfloat16)
    ones_col = jnp.ones((n, 1), jnp.bfloat16)
    rhs_aug = jnp.concatenate([xtb, ones_col], axis=1)
    outs = []
    for p in range(2):
        y1 = y[:, p:p + 1]
        y2c = y[:, 2 + p:3 + p]
        y2 = jnp.transpose(y2c)
        y2max = jnp.max(y2c)
        vb = y1 + y2max
        mrow = jnp.maximum(vb, 0.2 * vb)
        c1 = (y1 - mrow) * _LOG2E
        c2 = (0.2 * y1 - mrow) * _LOG2E
        r1 = y2 * _LOG2E
        r2 = y2 * (0.2 * _LOG2E)
        arg = jnp.maximum(c1 + r1, c2 + r2)
        ex = jnp.exp2(arg) * maskf
        z_aug = jnp.dot(ex.astype(jnp.bfloat16), rhs_aug,
                        preferred_element_type=jnp.float32)
        recip = 1.0 / z_aug[:, g:g + 1]
        zb = z_aug[:, :g].astype(jnp.bfloat16)
        o = (jnp.dot(xtb, w0[p], preferred_element_type=jnp.float32)
             + recip * jnp.dot(zb, w1[p], preferred_element_type=jnp.float32)
             + brow)
        outs.append(jnp.maximum(o, 0.0))
    return jnp.concatenate(outs, axis=1)


def _body(x_ref, s_ref,
          mc0, w00, w10, b0,
          mc1, w01, w11, b1,
          mc2, w02, w12, b2,
          mc3, w03, w13, b3,
          mc4, w04, w14, b4,
          mc5, w05, w15, b5,
          out_ref):
    xt = jnp.transpose(x_ref[0])
    mask0 = (jnp.abs(s_ref[0, 0]) > 1e-9).astype(jnp.float32)
    mask1 = (jnp.abs(s_ref[0, 1]) > 1e-9).astype(jnp.float32)
    p1 = _layer(xt, mask0, mc0[...], w00[...], w10[...], b0[...])
    p2 = _layer(p1, mask1, mc1[...], w01[...], w11[...], b1[...])
    p3 = (_layer(p2, mask1, mc2[...], w02[...], w12[...], b2[...])
          + _layer(p1, mask1, mc5[...], w05[...], w15[...], b5[...]))
    p4 = (_layer(p3, mask0, mc3[...], w03[...], w13[...], b3[...])
          + _layer(xt, mask0, mc4[...], w04[...], w14[...], b4[...]))
    out_ref[0] = jnp.transpose(p4)


def _prep(mixer, weight, bias):
    g = mixer.shape[2] // 2
    a1 = mixer[:, 0, :g]
    a2 = mixer[:, 0, g:]
    mc = jnp.concatenate([a1, a2], axis=0).T
    w0 = weight[:, 0, 0].astype(jnp.bfloat16)
    w1 = weight[:, 0, 1].astype(jnp.bfloat16)
    brow = bias.T
    return mc, w0, w1, brow


@jax.jit
def kernel(x, Slist,
           down0_mixer, down0_weight, down0_bias,
           down1_mixer, down1_weight, down1_bias,
           up0_mixer, up0_weight, up0_bias,
           up1_mixer, up1_weight, up1_bias,
           sc0_mixer, sc0_weight, sc0_bias,
           sc1_mixer, sc1_weight, sc1_bias):
    B, Fin, N = x.shape

    params = []
    for m, w, b in ((down0_mixer, down0_weight, down0_bias),
                    (down1_mixer, down1_weight, down1_bias),
                    (up0_mixer, up0_weight, up0_bias),
                    (up1_mixer, up1_weight, up1_bias),
                    (sc0_mixer, sc0_weight, sc0_bias),
                    (sc1_mixer, sc1_weight, sc1_bias)):
        params.extend(_prep(m, w, b))

    full = lambda a: pl.BlockSpec(a.shape, lambda b: (0,) * a.ndim)
    in_specs = [
        pl.BlockSpec((1, Fin, N), lambda b: (b, 0, 0)),
        pl.BlockSpec((1, 2, N, N), lambda b: (b, 0, 0, 0)),
    ] + [full(p) for p in params]

    return pl.pallas_call(
        _body,
        grid=(B,),
        in_specs=in_specs,
        out_specs=pl.BlockSpec((1, 2 * Fin, N), lambda b: (b, 0, 0)),
        out_shape=jax.ShapeDtypeStruct((B, 2 * Fin, N), jnp.float32),
        compiler_params=pltpu.CompilerParams(
            dimension_semantics=("parallel",),
        ),
    )(x, Slist, *params)
